# baseline (speedup 1.0000x reference)
import jax
import jax.numpy as jnp
from jax import lax
from jax.experimental import pallas as pl
from jax.experimental.pallas import tpu as pltpu
from jax.experimental.pallas import tpu_sc as plsc

B = 4096
L = 50
LP = 56
EMB = 128
LANES = 16
NJ = EMB // LANES

NC = 2
NS = 16
NW = NC * NS
RPW = B // NW


def _body(x_hbm, lens_hbm, table_hbm, out_hbm,
          idx_v, lens_v, rows0, rows1, out_v, sem0, sem1):
    wid = lax.axis_index("s") * NC + lax.axis_index("c")
    base = wid * RPW

    pltpu.sync_copy(x_hbm.at[pl.ds(base, RPW)], idx_v)
    pltpu.sync_copy(lens_hbm.at[pl.ds(base, RPW)], lens_v)

    rows = (rows0, rows1)
    sems = (sem0, sem1)

    pltpu.async_copy(table_hbm.at[idx_v.at[0]], rows0, sem0)
    pltpu.async_copy(table_hbm.at[idx_v.at[1]], rows1, sem1)

    def group(g, carry):
        lens16 = lens_v[pl.ds(g * LANES, LANES)]
        for b in range(LANES):
            r = g * LANES + b
            rows_b = rows[b % 2]
            sem_b = sems[b % 2]
            pltpu.make_async_copy(
                table_hbm.at[pl.ds(0, LP)], rows_b, sem_b).wait()

            len_r = lens16[b]

            def acc_step(l, acc, rows_b=rows_b):
                return tuple(
                    acc[j] + rows_b[l, pl.ds(LANES * j, LANES)]
                    for j in range(NJ))

            zeros = tuple(jnp.zeros((LANES,), jnp.float32)
                          for _ in range(NJ))
            acc = lax.fori_loop(0, len_r, acc_step, zeros)

            len_f = jnp.full((LANES,), len_r.astype(jnp.float32))
            scale = jnp.where(
                len_r > 0, jnp.full((LANES,), 1.0) / len_f,
                jnp.zeros((LANES,)))
            for j in range(NJ):
                out_v[r, pl.ds(LANES * j, LANES)] = acc[j] * scale

            @pl.when(r + 2 < RPW)
            def _(rows_b=rows_b, sem_b=sem_b, r=r):
                pltpu.async_copy(
                    table_hbm.at[idx_v.at[r + 2]], rows_b, sem_b)
        return carry

    lax.fori_loop(0, RPW // LANES, group, 0)

    pltpu.sync_copy(out_v, out_hbm.at[pl.ds(base, RPW)])


@jax.jit
def kernel(x, x_lens, table):
    xp = jnp.pad(x.astype(jnp.int32), ((0, 0), (0, LP - L)))
    lens = x_lens.astype(jnp.int32)

    mesh = plsc.VectorSubcoreMesh(
        core_axis_name="c", subcore_axis_name="s",
        num_cores=NC, num_subcores=NS)
    f = pl.kernel(
        _body,
        out_type=jax.ShapeDtypeStruct((B, EMB), jnp.float32),
        mesh=mesh,
        scratch_types=[
            pltpu.VMEM((RPW, LP), jnp.int32),
            pltpu.VMEM((RPW,), jnp.int32),
            pltpu.VMEM((LP, EMB), jnp.float32),
            pltpu.VMEM((LP, EMB), jnp.float32),
            pltpu.VMEM((RPW, EMB), jnp.float32),
            pltpu.SemaphoreType.DMA,
            pltpu.SemaphoreType.DMA,
        ],
    )
    return f(xp, lens, table)
